# baseline (speedup 1.0000x reference)
import functools

import jax
import jax.numpy as jnp
from jax import lax
from jax.experimental import pallas as pl
from jax.experimental.pallas import tpu as pltpu
from jax.experimental.pallas import tpu_sc as plsc

VOCAB = 64
D = 128
BATCH = 4096
HIST = 50
TOTAL = BATCH * HIST
NC = 2
NS = 16
NW = NC * NS
PER_W = TOTAL // NW
CH = 256
NCH = PER_W // CH
L = 16

_DNUMS = lax.GatherDimensionNumbers(
    offset_dims=(), collapsed_slice_dims=(0,), start_index_map=(0,))


def _splat(vec, j):
    return lax.gather(vec, jnp.full((L, 1), j, jnp.int32), _DNUMS, (1,),
                      mode=lax.GatherScatterMode.PROMISE_IN_BOUNDS)


@functools.partial(
    pl.kernel,
    out_type=jax.ShapeDtypeStruct((TOTAL, D), jnp.float32),
    mesh=plsc.VectorSubcoreMesh(core_axis_name="c", subcore_axis_name="s"),
    compiler_params=pltpu.CompilerParams(needs_layout_passes=False),
    scratch_types=[
        pltpu.VMEM_SHARED((VOCAB, D), jnp.float32),
        pltpu.VMEM((PER_W,), jnp.int32),
        pltpu.VMEM((PER_W,), jnp.float32),
        pltpu.VMEM((3, CH, D), jnp.float32),
        pltpu.SemaphoreType.DMA,
        pltpu.SemaphoreType.DMA,
    ],
)
def _lookup_scale(idx_hbm, scale_hbm, table_hbm, out_hbm,
                  table_sh, idx_v, scale_v, rows_v, sem_g, sem_o):
    sid = lax.axis_index("s")
    wid = sid * NC + lax.axis_index("c")
    base = pl.multiple_of(wid * PER_W, CH)

    @pl.when(sid == 0)
    def _():
        pltpu.sync_copy(table_hbm, table_sh)

    pltpu.sync_copy(idx_hbm.at[pl.ds(base, PER_W)], idx_v)
    pltpu.sync_copy(scale_hbm.at[pl.ds(base, PER_W)], scale_v)
    plsc.subcore_barrier()

    def gather(c, buf):
        pltpu.async_copy(
            table_sh.at[idx_v.at[pl.ds(c * CH, CH)]], rows_v.at[buf], sem_g)

    def scale_chunk(c, buf):
        def group_body(g2, _):
            for h in range(2):
                r0 = (g2 * 2 + h) * L
                sv = scale_v[pl.ds(c * CH + r0, L)]
                for j in range(L):
                    i = r0 + j
                    sj = _splat(sv, j)
                    for d8 in range(D // L):
                        sl = pl.ds(d8 * L, L)
                        rows_v[buf, i, sl] = rows_v[buf, i, sl] * sj
            return 0

        lax.fori_loop(0, CH // L // 2, group_body, 0)

    def wait_gather(buf):
        pltpu.make_async_copy(
            table_sh.at[idx_v.at[pl.ds(0, CH)]], rows_v.at[buf], sem_g).wait()

    def out_start(c, buf):
        off = pl.multiple_of(base + c * CH, CH)
        pltpu.async_copy(rows_v.at[buf], out_hbm.at[pl.ds(off, CH)], sem_o)

    def out_wait(buf):
        pltpu.make_async_copy(
            rows_v.at[buf], out_hbm.at[pl.ds(0, CH)], sem_o).wait()

    NBUF = 3
    gather(0, 0)
    for c in range(NCH):
        buf = c % NBUF
        nbuf = (c + 1) % NBUF
        if c < NCH - 1:
            if c + 1 >= NBUF:
                out_wait(nbuf)
            gather(c + 1, nbuf)
        wait_gather(buf)
        scale_chunk(c, buf)
        out_start(c, buf)
    for b in range(NBUF):
        out_wait(b)


def kernel(x_indices, x_scale, table):
    idx = x_indices.reshape(TOTAL).astype(jnp.int32)
    scale = x_scale.reshape(TOTAL)
    out = _lookup_scale(idx, scale, table)
    return out.reshape(BATCH, HIST, D)


# speedup vs baseline: 1.0024x; 1.0024x over previous
import functools

import jax
import jax.numpy as jnp
from jax import lax
from jax.experimental import pallas as pl
from jax.experimental.pallas import tpu as pltpu
from jax.experimental.pallas import tpu_sc as plsc

VOCAB = 64
D = 128
BATCH = 4096
HIST = 50
TOTAL = BATCH * HIST
NC = 2
NS = 16
NW = NC * NS
PER_W = TOTAL // NW
CH = 160
NCH = PER_W // CH
L = 16

_DNUMS = lax.GatherDimensionNumbers(
    offset_dims=(), collapsed_slice_dims=(0,), start_index_map=(0,))


def _splat(vec, j):
    return lax.gather(vec, jnp.full((L, 1), j, jnp.int32), _DNUMS, (1,),
                      mode=lax.GatherScatterMode.PROMISE_IN_BOUNDS)


@functools.partial(
    pl.kernel,
    out_type=jax.ShapeDtypeStruct((TOTAL, D), jnp.float32),
    mesh=plsc.VectorSubcoreMesh(core_axis_name="c", subcore_axis_name="s"),
    compiler_params=pltpu.CompilerParams(needs_layout_passes=False),
    scratch_types=[
        pltpu.VMEM_SHARED((VOCAB, D), jnp.float32),
        pltpu.VMEM((PER_W,), jnp.int32),
        pltpu.VMEM((PER_W,), jnp.float32),
        pltpu.VMEM((4, CH, D), jnp.float32),
        pltpu.SemaphoreType.DMA,
        pltpu.SemaphoreType.DMA,
    ],
)
def _lookup_scale(idx_hbm, scale_hbm, table_hbm, out_hbm,
                  table_sh, idx_v, scale_v, rows_v, sem_g, sem_o):
    sid = lax.axis_index("s")
    wid = sid * NC + lax.axis_index("c")
    base = pl.multiple_of(wid * PER_W, 8)

    @pl.when(sid == 0)
    def _():
        pltpu.sync_copy(table_hbm, table_sh)

    pltpu.sync_copy(idx_hbm.at[pl.ds(base, PER_W)], idx_v)
    pltpu.sync_copy(scale_hbm.at[pl.ds(base, PER_W)], scale_v)
    plsc.subcore_barrier()

    def gather(c, buf):
        pltpu.async_copy(
            table_sh.at[idx_v.at[pl.ds(c * CH, CH)]], rows_v.at[buf], sem_g)

    def scale_chunk(c, buf):
        def group_body(g, _):
            r0 = g * L
            sv = scale_v[pl.ds(c * CH + r0, L)]
            for j in range(L):
                i = r0 + j
                sj = _splat(sv, j)
                for d8 in range(D // L):
                    sl = pl.ds(d8 * L, L)
                    rows_v[buf, i, sl] = rows_v[buf, i, sl] * sj
            return 0

        lax.fori_loop(0, CH // L, group_body, 0)

    def wait_gather(buf):
        pltpu.make_async_copy(
            table_sh.at[idx_v.at[pl.ds(0, CH)]], rows_v.at[buf], sem_g).wait()

    def out_start(c, buf):
        off = pl.multiple_of(base + c * CH, CH)
        pltpu.async_copy(rows_v.at[buf], out_hbm.at[pl.ds(off, CH)], sem_o)

    def out_wait(buf):
        pltpu.make_async_copy(
            rows_v.at[buf], out_hbm.at[pl.ds(0, CH)], sem_o).wait()

    NBUF = 4
    gather(0, 0)
    for c in range(NCH):
        buf = c % NBUF
        nbuf = (c + 1) % NBUF
        if c < NCH - 1:
            if c + 1 >= NBUF:
                out_wait(nbuf)
            gather(c + 1, nbuf)
        wait_gather(buf)
        scale_chunk(c, buf)
        out_start(c, buf)
    for b in range(NBUF):
        out_wait(b)


def kernel(x_indices, x_scale, table):
    idx = x_indices.reshape(TOTAL).astype(jnp.int32)
    scale = x_scale.reshape(TOTAL)
    out = _lookup_scale(idx, scale, table)
    return out.reshape(BATCH, HIST, D)


# speedup vs baseline: 1.0168x; 1.0144x over previous
import functools

import jax
import jax.numpy as jnp
from jax import lax
from jax.experimental import pallas as pl
from jax.experimental.pallas import tpu as pltpu
from jax.experimental.pallas import tpu_sc as plsc

VOCAB = 64
D = 128
BATCH = 4096
HIST = 50
TOTAL = BATCH * HIST
NC = 2
NS = 16
NW = NC * NS
PER_W = TOTAL // NW
CH = 256
NCH = PER_W // CH
L = 16

_DNUMS = lax.GatherDimensionNumbers(
    offset_dims=(), collapsed_slice_dims=(0,), start_index_map=(0,))


def _splat(vec, j):
    return lax.gather(vec, jnp.full((L, 1), j, jnp.int32), _DNUMS, (1,),
                      mode=lax.GatherScatterMode.PROMISE_IN_BOUNDS)


@functools.partial(
    pl.kernel,
    out_type=jax.ShapeDtypeStruct((TOTAL, D), jnp.float32),
    mesh=plsc.VectorSubcoreMesh(core_axis_name="c", subcore_axis_name="s"),
    compiler_params=pltpu.CompilerParams(needs_layout_passes=False),
    scratch_types=[
        pltpu.VMEM_SHARED((VOCAB, D), jnp.float32),
        pltpu.VMEM((PER_W,), jnp.int32),
        pltpu.VMEM((PER_W,), jnp.float32),
        pltpu.VMEM((3, CH, D), jnp.float32),
        pltpu.SemaphoreType.DMA,
        pltpu.SemaphoreType.DMA,
    ],
)
def _lookup_scale(idx_hbm, scale_hbm, table_hbm, out_hbm,
                  table_sh, idx_v, scale_v, rows_v, sem_g, sem_o):
    sid = lax.axis_index("s")
    wid = sid * NC + lax.axis_index("c")
    base = pl.multiple_of(wid * PER_W, CH)

    pltpu.async_copy(idx_hbm.at[pl.ds(base, PER_W)], idx_v, sem_g)
    pltpu.async_copy(scale_hbm.at[pl.ds(base, PER_W)], scale_v, sem_g)

    @pl.when(sid == 0)
    def _():
        pltpu.sync_copy(table_hbm, table_sh)

    pltpu.make_async_copy(idx_hbm.at[pl.ds(base, PER_W)], idx_v, sem_g).wait()
    pltpu.make_async_copy(
        scale_hbm.at[pl.ds(base, PER_W)], scale_v, sem_g).wait()
    plsc.subcore_barrier()

    def gather(c, buf):
        pltpu.async_copy(
            table_sh.at[idx_v.at[pl.ds(c * CH, CH)]], rows_v.at[buf], sem_g)

    def scale_chunk(c, buf):
        def group_body(g, _):
            r0 = g * L
            sv = scale_v[pl.ds(c * CH + r0, L)]
            for j in range(L):
                i = r0 + j
                sj = _splat(sv, j)
                for d8 in range(D // L):
                    sl = pl.ds(d8 * L, L)
                    rows_v[buf, i, sl] = rows_v[buf, i, sl] * sj
            return 0

        lax.fori_loop(0, CH // L, group_body, 0)

    def wait_gather(buf):
        pltpu.make_async_copy(
            table_sh.at[idx_v.at[pl.ds(0, CH)]], rows_v.at[buf], sem_g).wait()

    def out_start(c, buf):
        off = pl.multiple_of(base + c * CH, CH)
        pltpu.async_copy(rows_v.at[buf], out_hbm.at[pl.ds(off, CH)], sem_o)

    def out_wait(buf):
        pltpu.make_async_copy(
            rows_v.at[buf], out_hbm.at[pl.ds(0, CH)], sem_o).wait()

    NBUF = 3
    gather(0, 0)
    for c in range(NCH):
        buf = c % NBUF
        nbuf = (c + 1) % NBUF
        if c < NCH - 1:
            if c + 1 >= NBUF:
                out_wait(nbuf)
            gather(c + 1, nbuf)
        wait_gather(buf)
        scale_chunk(c, buf)
        out_start(c, buf)
    for b in range(NBUF):
        out_wait(b)


def kernel(x_indices, x_scale, table):
    idx = x_indices.reshape(TOTAL).astype(jnp.int32)
    scale = x_scale.reshape(TOTAL)
    out = _lookup_scale(idx, scale, table)
    return out.reshape(BATCH, HIST, D)
